# baseline (speedup 1.0000x reference)
import functools

import jax
import jax.numpy as jnp
from jax import lax
from jax.experimental import pallas as pl
from jax.experimental.pallas import tpu as pltpu
from jax.experimental.pallas import tpu_sc as plsc

_EMB = 64
_SCALE = 8.0
_NC, _NS, _L = 2, 16, 16
_NW = _NC * _NS
_B = 4096 * 200
_BPW = _B // _NW
_C = 512
_NCHUNK = _BPW // _C

_mesh = plsc.VectorSubcoreMesh(core_axis_name="c", subcore_axis_name="s")


@functools.partial(
    pl.kernel,
    out_type=jax.ShapeDtypeStruct((_B, _EMB), jnp.float32),
    mesh=_mesh,
    scratch_types=[
        pltpu.VMEM((_BPW,), jnp.int32),
        pltpu.VMEM((_C, _EMB), jnp.float32),
        pltpu.SemaphoreType.DMA,
        pltpu.SemaphoreType.DMA,
    ],
    compiler_params=pltpu.CompilerParams(use_tc_tiling_on_sc=False),
)
def _emb_lookup(table_hbm, idx_hbm, out_hbm, idx_v, rows_v, gsem, osem):
    wid = lax.axis_index("s") * _NC + lax.axis_index("c")
    base = wid * _BPW
    pltpu.sync_copy(idx_hbm.at[pl.ds(base, _BPW)], idx_v)

    @pl.loop(0, _NCHUNK)
    def _chunk(k):
        pltpu.async_copy(
            table_hbm.at[idx_v.at[pl.ds(k * _C, _C)]], rows_v, gsem
        ).wait()

        @pl.loop(0, _C, unroll=4)
        def _row(i):
            for j in range(_EMB // _L):
                sl = pl.ds(j * _L, _L)
                rows_v[i, sl] = rows_v[i, sl] * _SCALE

        pltpu.async_copy(
            rows_v, out_hbm.at[pl.ds(base + k * _C, _C)], osem
        ).wait()


def kernel(tokens, table):
    idx = tokens.reshape(-1)
    out = _emb_lookup(table, idx)
    return out.reshape(*tokens.shape, _EMB)
